# Initial kernel scaffold; baseline (speedup 1.0000x reference)
import jax
import jax.numpy as jnp
from jax.experimental import pallas as pl


def kernel(x1, x2, CG_vals, M1, M2, M_out):
    raise NotImplementedError("write your pallas kernel here")


# speedup vs baseline: 3.9311x; 3.9311x over previous
import jax
import jax.numpy as jnp
from jax.experimental import pallas as pl
from jax.experimental.pallas import tpu as pltpu

_TN = 200


def _body(w_ref, x1_ref, x2_ref, o_ref):
    no = w_ref.shape[0]
    c = x1_ref.shape[1] // no
    a = [x1_ref[:, m * c:(m + 1) * c] for m in range(no)]
    b = [x2_ref[:, m * c:(m + 1) * c] for m in range(no)]
    p = [[a[m1] * b[m2] for m2 in range(no)] for m1 in range(no)]
    for m in range(no):
        acc = jnp.zeros_like(p[0][0])
        for m1 in range(no):
            for m2 in range(no):
                acc = acc + w_ref[m1, m2, m] * p[m1][m2]
        o_ref[:, m * c:(m + 1) * c] = acc


def kernel(x1, x2, CG_vals, M1, M2, M_out):
    n, no, c = x1.shape
    w = jnp.zeros((no, no, no), jnp.float32).at[M1, M2, M_out].add(CG_vals)
    x1f = x1.reshape(n, no * c)
    x2f = x2.reshape(n, no * c)
    grid = n // _TN
    out = pl.pallas_call(
        _body,
        grid=(grid,),
        in_specs=[
            pl.BlockSpec(memory_space=pltpu.SMEM),
            pl.BlockSpec((_TN, no * c), lambda i: (i, 0)),
            pl.BlockSpec((_TN, no * c), lambda i: (i, 0)),
        ],
        out_specs=pl.BlockSpec((_TN, no * c), lambda i: (i, 0)),
        out_shape=jax.ShapeDtypeStruct((n, no * c), x1.dtype),
        compiler_params=pltpu.CompilerParams(
            dimension_semantics=("arbitrary",)),
    )(w, x1f, x2f)
    return out.reshape(n, no, c)
